# baseline (speedup 1.0000x reference)
import functools

import jax
import jax.numpy as jnp
from jax import lax
from jax.experimental import pallas as pl
from jax.experimental.pallas import tpu as pltpu
from jax.experimental.pallas import tpu_sc as plsc

EPS = 1e-05

N = 10000
D = 128
E = 320000
NP = 10240
L = 16

NC, NS = 2, 16
TILES = NC * NS
EPT = E // TILES
CH = 80
NCH = EPT // CH


def _stats_body(x_ref, s_ref, q_ref):
    xb = x_ref[...]
    s_ref[...] = jnp.mean(xb, axis=1, keepdims=True)
    q_ref[...] = jnp.mean(xb * xb, axis=1, keepdims=True)


def _row_stats(x):
    blk = 2000
    s, q = pl.pallas_call(
        _stats_body,
        grid=(N // blk,),
        in_specs=[pl.BlockSpec((blk, D), lambda i: (i, 0))],
        out_specs=[pl.BlockSpec((blk, 1), lambda i: (i, 0)),
                   pl.BlockSpec((blk, 1), lambda i: (i, 0))],
        out_shape=[jax.ShapeDtypeStruct((N, 1), jnp.float32),
                   jax.ShapeDtypeStruct((N, 1), jnp.float32)],
    )(x)
    return s.reshape(N), q.reshape(N)


def _scatter_body(s_hbm, q_hbm, col_hbm, row_hbm, parts, colv, rowv, sbuf,
                  qbuf, ones, zbuf, S_sh, Q_sh, D_sh, semg, sem_sc, sem_ones):
    cid = lax.axis_index("c")
    sid = lax.axis_index("s")
    wid = cid * NS + sid

    pltpu.sync_copy(col_hbm.at[wid], colv)
    pltpu.sync_copy(row_hbm.at[wid], rowv)

    for i in range(CH // L):
        ones[pl.ds(i * L, L)] = jnp.ones((L,), jnp.float32)

    @pl.when(sid < 3)
    def _():
        @pl.loop(0, NP // L)
        def _(i):
            zbuf[pl.ds(i * L, L)] = jnp.zeros((L,), jnp.float32)

    @pl.when(sid == 0)
    def _():
        pltpu.sync_copy(zbuf, S_sh)

    @pl.when(sid == 1)
    def _():
        pltpu.sync_copy(zbuf, Q_sh)

    @pl.when(sid == 2)
    def _():
        pltpu.sync_copy(zbuf, D_sh)

    plsc.subcore_barrier()

    DEPTH = 6
    NBUF = 8

    @pl.loop(0, NCH + DEPTH)
    def _(c):
        b = lax.rem(c, NBUF)

        @pl.when(c < NCH)
        def _():
            @pl.when(c >= NBUF)
            def _():
                pltpu.make_async_copy(
                    s_hbm.at[pl.ds(0, CH)], sbuf.at[b], sem_sc.at[b]).wait()
                pltpu.make_async_copy(
                    q_hbm.at[pl.ds(0, CH)], qbuf.at[b], sem_sc.at[b]).wait()
                pltpu.make_async_copy(
                    s_hbm.at[pl.ds(0, CH)], ones, sem_ones).wait()

            pltpu.async_copy(s_hbm.at[colv.at[c]], sbuf.at[b], semg.at[b])
            pltpu.async_copy(q_hbm.at[colv.at[c]], qbuf.at[b], semg.at[b])

        cs = c - DEPTH

        @pl.when(cs >= 0)
        def _():
            bs = lax.rem(cs, NBUF)
            pltpu.make_async_copy(
                s_hbm.at[pl.ds(0, CH)], sbuf.at[bs], semg.at[bs]).wait()
            pltpu.make_async_copy(
                q_hbm.at[pl.ds(0, CH)], qbuf.at[bs], semg.at[bs]).wait()
            pltpu.async_copy(sbuf.at[bs], S_sh.at[rowv.at[cs]],
                             sem_sc.at[bs], add=True)
            pltpu.async_copy(qbuf.at[bs], Q_sh.at[rowv.at[cs]],
                             sem_sc.at[bs], add=True)
            pltpu.async_copy(ones, D_sh.at[rowv.at[cs]], sem_ones, add=True)

    for b in range(NBUF):
        pltpu.make_async_copy(
            s_hbm.at[pl.ds(0, CH)], sbuf.at[b], sem_sc.at[b]).wait()
        pltpu.make_async_copy(
            q_hbm.at[pl.ds(0, CH)], qbuf.at[b], sem_sc.at[b]).wait()
        pltpu.make_async_copy(
            s_hbm.at[pl.ds(0, CH)], ones, sem_ones).wait()

    plsc.subcore_barrier()

    @pl.when(sid == 0)
    def _():
        pltpu.sync_copy(S_sh, parts.at[pl.ds((cid * 3 + 0) * NP, NP)])
        pltpu.sync_copy(Q_sh, parts.at[pl.ds((cid * 3 + 1) * NP, NP)])
        pltpu.sync_copy(D_sh, parts.at[pl.ds((cid * 3 + 2) * NP, NP)])


def _scatter_partials(s, q, col3, row3):
    mesh = plsc.VectorSubcoreMesh(core_axis_name="c", subcore_axis_name="s")
    f = pl.kernel(
        _scatter_body,
        out_type=jax.ShapeDtypeStruct((NC * 3 * NP,), jnp.float32),
        mesh=mesh,
        scratch_types=[
            pltpu.VMEM((NCH, CH), jnp.int32),
            pltpu.VMEM((NCH, CH), jnp.int32),
            pltpu.VMEM((8, CH), jnp.float32),
            pltpu.VMEM((8, CH), jnp.float32),
            pltpu.VMEM((CH,), jnp.float32),
            pltpu.VMEM((NP,), jnp.float32),
            pltpu.VMEM_SHARED((NP,), jnp.float32),
            pltpu.VMEM_SHARED((NP,), jnp.float32),
            pltpu.VMEM_SHARED((NP,), jnp.float32),
            pltpu.SemaphoreType.DMA((8,)),
            pltpu.SemaphoreType.DMA((8,)),
            pltpu.SemaphoreType.DMA,
        ],
    )
    return f(s, q, col3, row3)


NSL = NP // NS
NBF = 4


def _rsqrt16(xx):
    i = lax.bitcast_convert_type(xx, jnp.int32)
    y = lax.bitcast_convert_type(jnp.int32(0x5F3759DF) - (i >> 1), jnp.float32)
    for _ in range(4):
        y = y * (1.5 - 0.5 * xx * y * y)
    return y


def _main_body(x_hbm, col_hbm, row_hbm, parts, g_hbm, b_hbm,
               out_hbm, m_out, i_out,
               colv, rowv, mbuf, ivbuf, xbuf, obuf, gbuf, bbuf, ptmp,
               mtmp, itmp, semg, semo):
    cid = lax.axis_index("c")
    sid = lax.axis_index("s")
    wid = cid * NS + sid
    ebase = wid * EPT

    pltpu.sync_copy(col_hbm.at[wid], colv)
    pltpu.sync_copy(row_hbm.at[wid], rowv)
    pltpu.sync_copy(g_hbm, gbuf)
    pltpu.sync_copy(b_hbm, bbuf)

    nb = sid * NSL
    for t in range(6):
        c6, t3 = divmod(t, 3)
        pltpu.sync_copy(parts.at[pl.ds((c6 * 3 + t3) * NP + nb, NSL)],
                        ptmp.at[t])

    @pl.loop(0, NSL // L)
    def _(gi):
        sl = pl.ds(gi * L, L)
        S = ptmp[0, sl] + ptmp[3, sl]
        Q = ptmp[1, sl] + ptmp[4, sl]
        dg = ptmp[2, sl] + ptmp[5, sl]
        den = jnp.maximum(dg, 1.0)
        m = S / den
        var = jnp.maximum(Q / den - m * m, 0.0)
        mtmp[sl] = m
        itmp[sl] = _rsqrt16(var + EPS)

    pltpu.sync_copy(mtmp, m_out.at[pl.ds(cid * NP + nb, NSL)])
    pltpu.sync_copy(itmp, i_out.at[pl.ds(cid * NP + nb, NSL)])
    plsc.subcore_barrier()

    g_regs = [gbuf[pl.ds(d * L, L)] for d in range(D // L)]
    b_regs = [bbuf[pl.ds(d * L, L)] for d in range(D // L)]

    def issue(c, b):
        pltpu.async_copy(m_out.at[rowv.at[c]], mbuf.at[b], semg.at[b])
        pltpu.async_copy(i_out.at[rowv.at[c]], ivbuf.at[b], semg.at[b])
        pltpu.async_copy(x_hbm.at[colv.at[c]], xbuf.at[b], semg.at[b])

    for b in range(NBF):
        issue(b, b)

    @pl.loop(0, NCH + NBF - 1, step=NBF)
    def _(g):
        for b in range(NBF):
            c = g + b

            @pl.when(c < NCH)
            def _():
                pltpu.make_async_copy(
                    m_out.at[pl.ds(0, CH)], mbuf.at[b], semg.at[b]).wait()
                pltpu.make_async_copy(
                    m_out.at[pl.ds(0, CH)], ivbuf.at[b], semg.at[b]).wait()
                pltpu.make_async_copy(
                    x_hbm.at[pl.ds(0, CH)], xbuf.at[b], semg.at[b]).wait()

                @pl.when(c >= NBF)
                def _():
                    pltpu.make_async_copy(
                        obuf.at[b], out_hbm.at[pl.ds(0, CH)], semo.at[b]).wait()

                @pl.loop(0, CH // L)
                def _(gi):
                    mvec = mbuf[b, pl.ds(gi * L, L)]
                    ivvec = ivbuf[b, pl.ds(gi * L, L)]
                    for jj in range(L):
                        j = gi * L + jj
                        m = mvec[jj]
                        iv = ivvec[jj]
                        for d in range(D // L):
                            xv = xbuf[b, j, pl.ds(d * L, L)]
                            obuf[b, j, pl.ds(d * L, L)] = (
                                (xv - m) * iv * g_regs[d] + b_regs[d])

                pltpu.async_copy(
                    obuf.at[b], out_hbm.at[pl.ds(ebase + c * CH, CH)],
                    semo.at[b])

                @pl.when(c + NBF < NCH)
                def _():
                    issue(c + NBF, b)

    for b in range(NBF):
        pltpu.make_async_copy(
            obuf.at[b], out_hbm.at[pl.ds(0, CH)], semo.at[b]).wait()


def _main_pass(x, col3, row3, parts, gamma, beta):
    mesh = plsc.VectorSubcoreMesh(core_axis_name="c", subcore_axis_name="s")
    f = pl.kernel(
        _main_body,
        out_type=[jax.ShapeDtypeStruct((E, D), jnp.float32),
                  jax.ShapeDtypeStruct((NC * NP,), jnp.float32),
                  jax.ShapeDtypeStruct((NC * NP,), jnp.float32)],
        mesh=mesh,
        scratch_types=[
            pltpu.VMEM((NCH, CH), jnp.int32),
            pltpu.VMEM((NCH, CH), jnp.int32),
            pltpu.VMEM((NBF, CH), jnp.float32),
            pltpu.VMEM((NBF, CH), jnp.float32),
            pltpu.VMEM((NBF, CH, D), jnp.float32),
            pltpu.VMEM((NBF, CH, D), jnp.float32),
            pltpu.VMEM((D,), jnp.float32),
            pltpu.VMEM((D,), jnp.float32),
            pltpu.VMEM((6, NSL), jnp.float32),
            pltpu.VMEM((NSL,), jnp.float32),
            pltpu.VMEM((NSL,), jnp.float32),
            pltpu.SemaphoreType.DMA((NBF,)),
            pltpu.SemaphoreType.DMA((NBF,)),
        ],
    )
    return f(x, col3, row3, parts, gamma, beta)[0]


@jax.jit
def kernel(x, edge_index, gamma, beta):
    ei = edge_index.astype(jnp.int32)
    row3 = ei[0].reshape(TILES, NCH, CH)
    col3 = ei[1].reshape(TILES, NCH, CH)
    bias = (jnp.arange(TILES, dtype=jnp.int32) // NS * NP)[:, None, None]
    row3b = row3 + bias

    s, q = _row_stats(x)
    parts = _scatter_partials(s, q, col3, row3)
    return _main_pass(x, col3, row3b, parts, gamma, beta)


# speedup vs baseline: 1.0229x; 1.0229x over previous
import functools

import jax
import jax.numpy as jnp
from jax import lax
from jax.experimental import pallas as pl
from jax.experimental.pallas import tpu as pltpu
from jax.experimental.pallas import tpu_sc as plsc

EPS = 1e-05

N = 10000
D = 128
E = 320000
NP = 10240
L = 16

NC, NS = 2, 16
TILES = NC * NS
EPT = E // TILES
CH = 80
NCH = EPT // CH


def _stats_body(x_ref, s_ref, q_ref):
    xb = x_ref[...]
    s_ref[...] = jnp.mean(xb, axis=1, keepdims=True)
    q_ref[...] = jnp.mean(xb * xb, axis=1, keepdims=True)


def _row_stats(x):
    blk = 2000
    s, q = pl.pallas_call(
        _stats_body,
        grid=(N // blk,),
        in_specs=[pl.BlockSpec((blk, D), lambda i: (i, 0))],
        out_specs=[pl.BlockSpec((blk, 1), lambda i: (i, 0)),
                   pl.BlockSpec((blk, 1), lambda i: (i, 0))],
        out_shape=[jax.ShapeDtypeStruct((N, 1), jnp.float32),
                   jax.ShapeDtypeStruct((N, 1), jnp.float32)],
    )(x)
    return s.reshape(N), q.reshape(N)


def _scatter_body(s_hbm, q_hbm, col_hbm, row_hbm, parts, colv, rowv, sbuf,
                  qbuf, ones, zbuf, S_sh, Q_sh, D_sh, semg, sem_sc, sem_ones):
    cid = lax.axis_index("c")
    sid = lax.axis_index("s")
    wid = cid * NS + sid

    pltpu.sync_copy(col_hbm.at[wid], colv)
    pltpu.sync_copy(row_hbm.at[wid], rowv)

    for i in range(CH // L):
        ones[pl.ds(i * L, L)] = jnp.ones((L,), jnp.float32)

    @pl.when(sid < 3)
    def _():
        @pl.loop(0, NP // L)
        def _(i):
            zbuf[pl.ds(i * L, L)] = jnp.zeros((L,), jnp.float32)

    @pl.when(sid == 0)
    def _():
        pltpu.sync_copy(zbuf, S_sh)

    @pl.when(sid == 1)
    def _():
        pltpu.sync_copy(zbuf, Q_sh)

    @pl.when(sid == 2)
    def _():
        pltpu.sync_copy(zbuf, D_sh)

    plsc.subcore_barrier()

    DEPTH = 4
    NBUF = 8

    @pl.loop(0, NCH + DEPTH)
    def _(c):
        b = lax.rem(c, NBUF)

        @pl.when(c < NCH)
        def _():
            @pl.when(c >= NBUF)
            def _():
                pltpu.make_async_copy(
                    s_hbm.at[pl.ds(0, CH)], sbuf.at[b], sem_sc.at[b]).wait()
                pltpu.make_async_copy(
                    q_hbm.at[pl.ds(0, CH)], qbuf.at[b], sem_sc.at[b]).wait()
                pltpu.make_async_copy(
                    s_hbm.at[pl.ds(0, CH)], ones, sem_ones).wait()

            pltpu.async_copy(s_hbm.at[colv.at[c]], sbuf.at[b], semg.at[b])
            pltpu.async_copy(q_hbm.at[colv.at[c]], qbuf.at[b], semg.at[b])

        cs = c - DEPTH

        @pl.when(cs >= 0)
        def _():
            bs = lax.rem(cs, NBUF)
            pltpu.make_async_copy(
                s_hbm.at[pl.ds(0, CH)], sbuf.at[bs], semg.at[bs]).wait()
            pltpu.make_async_copy(
                q_hbm.at[pl.ds(0, CH)], qbuf.at[bs], semg.at[bs]).wait()
            pltpu.async_copy(sbuf.at[bs], S_sh.at[rowv.at[cs]],
                             sem_sc.at[bs], add=True)
            pltpu.async_copy(qbuf.at[bs], Q_sh.at[rowv.at[cs]],
                             sem_sc.at[bs], add=True)
            pltpu.async_copy(ones, D_sh.at[rowv.at[cs]], sem_ones, add=True)

    for b in range(NBUF):
        pltpu.make_async_copy(
            s_hbm.at[pl.ds(0, CH)], sbuf.at[b], sem_sc.at[b]).wait()
        pltpu.make_async_copy(
            q_hbm.at[pl.ds(0, CH)], qbuf.at[b], sem_sc.at[b]).wait()
        pltpu.make_async_copy(
            s_hbm.at[pl.ds(0, CH)], ones, sem_ones).wait()

    plsc.subcore_barrier()

    @pl.when(sid == 0)
    def _():
        pltpu.sync_copy(S_sh, parts.at[pl.ds((cid * 3 + 0) * NP, NP)])
        pltpu.sync_copy(Q_sh, parts.at[pl.ds((cid * 3 + 1) * NP, NP)])
        pltpu.sync_copy(D_sh, parts.at[pl.ds((cid * 3 + 2) * NP, NP)])


def _scatter_partials(s, q, col3, row3):
    mesh = plsc.VectorSubcoreMesh(core_axis_name="c", subcore_axis_name="s")
    f = pl.kernel(
        _scatter_body,
        out_type=jax.ShapeDtypeStruct((NC * 3 * NP,), jnp.float32),
        mesh=mesh,
        scratch_types=[
            pltpu.VMEM((NCH, CH), jnp.int32),
            pltpu.VMEM((NCH, CH), jnp.int32),
            pltpu.VMEM((8, CH), jnp.float32),
            pltpu.VMEM((8, CH), jnp.float32),
            pltpu.VMEM((CH,), jnp.float32),
            pltpu.VMEM((NP,), jnp.float32),
            pltpu.VMEM_SHARED((NP,), jnp.float32),
            pltpu.VMEM_SHARED((NP,), jnp.float32),
            pltpu.VMEM_SHARED((NP,), jnp.float32),
            pltpu.SemaphoreType.DMA((8,)),
            pltpu.SemaphoreType.DMA((8,)),
            pltpu.SemaphoreType.DMA,
        ],
    )
    return f(s, q, col3, row3)


NSL = NP // NS
NBF = 3


def _rsqrt16(xx):
    i = lax.bitcast_convert_type(xx, jnp.int32)
    y = lax.bitcast_convert_type(jnp.int32(0x5F3759DF) - (i >> 1), jnp.float32)
    for _ in range(4):
        y = y * (1.5 - 0.5 * xx * y * y)
    return y


def _main_body(x_hbm, col_hbm, row_hbm, parts, g_hbm, b_hbm,
               out_hbm, m_out, i_out,
               colv, rowv, mbuf, ivbuf, xbuf, obuf, gbuf, bbuf, ptmp,
               mtmp, itmp, semg, semo):
    cid = lax.axis_index("c")
    sid = lax.axis_index("s")
    wid = cid * NS + sid
    ebase = wid * EPT

    pltpu.sync_copy(col_hbm.at[wid], colv)
    pltpu.sync_copy(row_hbm.at[wid], rowv)
    pltpu.sync_copy(g_hbm, gbuf)
    pltpu.sync_copy(b_hbm, bbuf)

    nb = sid * NSL
    for t in range(6):
        c6, t3 = divmod(t, 3)
        pltpu.sync_copy(parts.at[pl.ds((c6 * 3 + t3) * NP + nb, NSL)],
                        ptmp.at[t])

    @pl.loop(0, NSL // L)
    def _(gi):
        sl = pl.ds(gi * L, L)
        S = ptmp[0, sl] + ptmp[3, sl]
        Q = ptmp[1, sl] + ptmp[4, sl]
        dg = ptmp[2, sl] + ptmp[5, sl]
        den = jnp.maximum(dg, 1.0)
        m = S / den
        var = jnp.maximum(Q / den - m * m, 0.0)
        mtmp[sl] = m
        itmp[sl] = _rsqrt16(var + EPS)

    pltpu.sync_copy(mtmp, m_out.at[pl.ds(cid * NP + nb, NSL)])
    pltpu.sync_copy(itmp, i_out.at[pl.ds(cid * NP + nb, NSL)])
    plsc.subcore_barrier()

    g_regs = [gbuf[pl.ds(d * L, L)] for d in range(D // L)]
    b_regs = [bbuf[pl.ds(d * L, L)] for d in range(D // L)]

    def issue(c, b):
        pltpu.async_copy(m_out.at[rowv.at[c]], mbuf.at[b], semg.at[b])
        pltpu.async_copy(i_out.at[rowv.at[c]], ivbuf.at[b], semg.at[b])
        pltpu.async_copy(x_hbm.at[colv.at[c]], xbuf.at[b], semg.at[b])

    for b in range(NBF):
        issue(b, b)

    @pl.loop(0, NCH + NBF - 1, step=NBF)
    def _(g):
        for b in range(NBF):
            c = g + b

            @pl.when(c < NCH)
            def _():
                pltpu.make_async_copy(
                    m_out.at[pl.ds(0, CH)], mbuf.at[b], semg.at[b]).wait()
                pltpu.make_async_copy(
                    m_out.at[pl.ds(0, CH)], ivbuf.at[b], semg.at[b]).wait()
                pltpu.make_async_copy(
                    x_hbm.at[pl.ds(0, CH)], xbuf.at[b], semg.at[b]).wait()

                @pl.when(c >= NBF)
                def _():
                    pltpu.make_async_copy(
                        obuf.at[b], out_hbm.at[pl.ds(0, CH)], semo.at[b]).wait()

                @pl.loop(0, CH // L)
                def _(gi):
                    mvec = mbuf[b, pl.ds(gi * L, L)]
                    ivvec = ivbuf[b, pl.ds(gi * L, L)]
                    for jj in range(L):
                        j = gi * L + jj
                        m = mvec[jj]
                        iv = ivvec[jj]
                        for d in range(D // L):
                            xv = xbuf[b, j, pl.ds(d * L, L)]
                            obuf[b, j, pl.ds(d * L, L)] = (
                                (xv - m) * iv * g_regs[d] + b_regs[d])

                pltpu.async_copy(
                    obuf.at[b], out_hbm.at[pl.ds(ebase + c * CH, CH)],
                    semo.at[b])

                @pl.when(c + NBF < NCH)
                def _():
                    issue(c + NBF, b)

    for b in range(NBF):
        pltpu.make_async_copy(
            obuf.at[b], out_hbm.at[pl.ds(0, CH)], semo.at[b]).wait()


def _main_pass(x, col3, row3, parts, gamma, beta):
    mesh = plsc.VectorSubcoreMesh(core_axis_name="c", subcore_axis_name="s")
    f = pl.kernel(
        _main_body,
        out_type=[jax.ShapeDtypeStruct((E, D), jnp.float32),
                  jax.ShapeDtypeStruct((NC * NP,), jnp.float32),
                  jax.ShapeDtypeStruct((NC * NP,), jnp.float32)],
        mesh=mesh,
        scratch_types=[
            pltpu.VMEM((NCH, CH), jnp.int32),
            pltpu.VMEM((NCH, CH), jnp.int32),
            pltpu.VMEM((NBF, CH), jnp.float32),
            pltpu.VMEM((NBF, CH), jnp.float32),
            pltpu.VMEM((NBF, CH, D), jnp.float32),
            pltpu.VMEM((NBF, CH, D), jnp.float32),
            pltpu.VMEM((D,), jnp.float32),
            pltpu.VMEM((D,), jnp.float32),
            pltpu.VMEM((6, NSL), jnp.float32),
            pltpu.VMEM((NSL,), jnp.float32),
            pltpu.VMEM((NSL,), jnp.float32),
            pltpu.SemaphoreType.DMA((NBF,)),
            pltpu.SemaphoreType.DMA((NBF,)),
        ],
    )
    return f(x, col3, row3, parts, gamma, beta)[0]


@jax.jit
def kernel(x, edge_index, gamma, beta):
    ei = edge_index.astype(jnp.int32)
    row3 = ei[0].reshape(TILES, NCH, CH)
    col3 = ei[1].reshape(TILES, NCH, CH)
    bias = (jnp.arange(TILES, dtype=jnp.int32) // NS * NP)[:, None, None]
    row3b = row3 + bias

    s, q = _row_stats(x)
    parts = _scatter_partials(s, q, col3, row3)
    return _main_pass(x, col3, row3b, parts, gamma, beta)


# speedup vs baseline: 1.0810x; 1.0568x over previous
import functools

import jax
import jax.numpy as jnp
from jax import lax
from jax.experimental import pallas as pl
from jax.experimental.pallas import tpu as pltpu
from jax.experimental.pallas import tpu_sc as plsc

EPS = 1e-05

N = 10000
D = 128
E = 320000
NP = 10240
L = 16

NC, NS = 2, 16
TILES = NC * NS
EPT = E // TILES
CH = 80
NCH = EPT // CH


def _stats_body(x_ref, s_ref, q_ref):
    xb = x_ref[...]
    s_ref[...] = jnp.mean(xb, axis=1, keepdims=True)
    q_ref[...] = jnp.mean(xb * xb, axis=1, keepdims=True)


def _row_stats(x):
    blk = 2000
    s, q = pl.pallas_call(
        _stats_body,
        grid=(N // blk,),
        in_specs=[pl.BlockSpec((blk, D), lambda i: (i, 0))],
        out_specs=[pl.BlockSpec((blk, 1), lambda i: (i, 0)),
                   pl.BlockSpec((blk, 1), lambda i: (i, 0))],
        out_shape=[jax.ShapeDtypeStruct((N, 1), jnp.float32),
                   jax.ShapeDtypeStruct((N, 1), jnp.float32)],
    )(x)
    return s.reshape(N), q.reshape(N)


def _scatter_body(xt_hbm, col_hbm, row_hbm, parts, s_out, q_out,
                  colv, rowv, sbuf, qbuf, ones, zbuf, xtt, stmp, qtmp,
                  S_sh, Q_sh, D_sh, semg, sem_sc, sem_ones):
    cid = lax.axis_index("c")
    sid = lax.axis_index("s")
    wid = cid * NS + sid

    pltpu.sync_copy(col_hbm.at[wid], colv)
    pltpu.sync_copy(row_hbm.at[wid], rowv)

    for i in range(CH // L):
        ones[pl.ds(i * L, L)] = jnp.ones((L,), jnp.float32)

    rb = sid * NSL
    rinv = jnp.float32(1.0 / D)

    for k in range(NSL // D):
        start = rb + k * D
        pltpu.sync_copy(xt_hbm.at[:, pl.ds(start, D)], xtt)

        @pl.loop(0, D // L)
        def _(gi):
            sl = pl.ds(gi * L, L)
            acc_s = jnp.zeros((L,), jnp.float32)
            acc_q = jnp.zeros((L,), jnp.float32)
            for d in range(D):
                v = xtt[d, sl]
                acc_s = acc_s + v
                acc_q = acc_q + v * v
            stmp[pl.ds(k * D + gi * L, L)] = acc_s * rinv
            qtmp[pl.ds(k * D + gi * L, L)] = acc_q * rinv

    pltpu.sync_copy(stmp, s_out.at[pl.ds(cid * NP + rb, NSL)])
    pltpu.sync_copy(qtmp, q_out.at[pl.ds(cid * NP + rb, NSL)])

    @pl.when(sid < 3)
    def _():
        @pl.loop(0, NP // L)
        def _(i):
            zbuf[pl.ds(i * L, L)] = jnp.zeros((L,), jnp.float32)

    @pl.when(sid == 0)
    def _():
        pltpu.sync_copy(zbuf, S_sh)

    @pl.when(sid == 1)
    def _():
        pltpu.sync_copy(zbuf, Q_sh)

    @pl.when(sid == 2)
    def _():
        pltpu.sync_copy(zbuf, D_sh)

    plsc.subcore_barrier()

    DEPTH = 4
    NBUF = 8

    @pl.loop(0, NCH + DEPTH)
    def _(c):
        b = lax.rem(c, NBUF)

        @pl.when(c < NCH)
        def _():
            @pl.when(c >= NBUF)
            def _():
                pltpu.make_async_copy(
                    s_out.at[pl.ds(0, CH)], sbuf.at[b], sem_sc.at[b]).wait()
                pltpu.make_async_copy(
                    s_out.at[pl.ds(0, CH)], qbuf.at[b], sem_sc.at[b]).wait()
                pltpu.make_async_copy(
                    s_out.at[pl.ds(0, CH)], ones, sem_ones).wait()

            pltpu.async_copy(s_out.at[colv.at[c]], sbuf.at[b], semg.at[b])
            pltpu.async_copy(q_out.at[colv.at[c]], qbuf.at[b], semg.at[b])

        cs = c - DEPTH

        @pl.when(cs >= 0)
        def _():
            bs = lax.rem(cs, NBUF)
            pltpu.make_async_copy(
                s_out.at[pl.ds(0, CH)], sbuf.at[bs], semg.at[bs]).wait()
            pltpu.make_async_copy(
                s_out.at[pl.ds(0, CH)], qbuf.at[bs], semg.at[bs]).wait()
            pltpu.async_copy(sbuf.at[bs], S_sh.at[rowv.at[cs]],
                             sem_sc.at[bs], add=True)
            pltpu.async_copy(qbuf.at[bs], Q_sh.at[rowv.at[cs]],
                             sem_sc.at[bs], add=True)
            pltpu.async_copy(ones, D_sh.at[rowv.at[cs]], sem_ones, add=True)

    for b in range(NBUF):
        pltpu.make_async_copy(
            s_out.at[pl.ds(0, CH)], sbuf.at[b], sem_sc.at[b]).wait()
        pltpu.make_async_copy(
            s_out.at[pl.ds(0, CH)], qbuf.at[b], sem_sc.at[b]).wait()
        pltpu.make_async_copy(
            s_out.at[pl.ds(0, CH)], ones, sem_ones).wait()

    plsc.subcore_barrier()

    @pl.when(sid == 0)
    def _():
        pltpu.sync_copy(S_sh, parts.at[pl.ds((cid * 3 + 0) * NP, NP)])
        pltpu.sync_copy(Q_sh, parts.at[pl.ds((cid * 3 + 1) * NP, NP)])
        pltpu.sync_copy(D_sh, parts.at[pl.ds((cid * 3 + 2) * NP, NP)])


def _scatter_partials(xt, col3b, row3):
    mesh = plsc.VectorSubcoreMesh(core_axis_name="c", subcore_axis_name="s")
    f = pl.kernel(
        _scatter_body,
        out_type=[jax.ShapeDtypeStruct((NC * 3 * NP,), jnp.float32),
                  jax.ShapeDtypeStruct((NC * NP,), jnp.float32),
                  jax.ShapeDtypeStruct((NC * NP,), jnp.float32)],
        mesh=mesh,
        scratch_types=[
            pltpu.VMEM((NCH, CH), jnp.int32),
            pltpu.VMEM((NCH, CH), jnp.int32),
            pltpu.VMEM((8, CH), jnp.float32),
            pltpu.VMEM((8, CH), jnp.float32),
            pltpu.VMEM((CH,), jnp.float32),
            pltpu.VMEM((NP,), jnp.float32),
            pltpu.VMEM((D, D), jnp.float32),
            pltpu.VMEM((NSL,), jnp.float32),
            pltpu.VMEM((NSL,), jnp.float32),
            pltpu.VMEM_SHARED((NP,), jnp.float32),
            pltpu.VMEM_SHARED((NP,), jnp.float32),
            pltpu.VMEM_SHARED((NP,), jnp.float32),
            pltpu.SemaphoreType.DMA((8,)),
            pltpu.SemaphoreType.DMA((8,)),
            pltpu.SemaphoreType.DMA,
        ],
    )
    return f(xt, col3b, row3)[0]


NSL = NP // NS
NBF = 3


def _rsqrt16(xx):
    i = lax.bitcast_convert_type(xx, jnp.int32)
    y = lax.bitcast_convert_type(jnp.int32(0x5F3759DF) - (i >> 1), jnp.float32)
    for _ in range(4):
        y = y * (1.5 - 0.5 * xx * y * y)
    return y


def _main_body(x_hbm, col_hbm, row_hbm, parts, g_hbm, b_hbm,
               out_hbm, m_out, i_out,
               colv, rowv, mbuf, ivbuf, xbuf, obuf, gbuf, bbuf, ptmp,
               mtmp, itmp, semg, semo):
    cid = lax.axis_index("c")
    sid = lax.axis_index("s")
    wid = cid * NS + sid
    ebase = wid * EPT

    pltpu.sync_copy(col_hbm.at[wid], colv)
    pltpu.sync_copy(row_hbm.at[wid], rowv)
    pltpu.sync_copy(g_hbm, gbuf)
    pltpu.sync_copy(b_hbm, bbuf)

    nb = sid * NSL
    for t in range(6):
        c6, t3 = divmod(t, 3)
        pltpu.sync_copy(parts.at[pl.ds((c6 * 3 + t3) * NP + nb, NSL)],
                        ptmp.at[t])

    @pl.loop(0, NSL // L)
    def _(gi):
        sl = pl.ds(gi * L, L)
        S = ptmp[0, sl] + ptmp[3, sl]
        Q = ptmp[1, sl] + ptmp[4, sl]
        dg = ptmp[2, sl] + ptmp[5, sl]
        den = jnp.maximum(dg, 1.0)
        m = S / den
        var = jnp.maximum(Q / den - m * m, 0.0)
        mtmp[sl] = m
        itmp[sl] = _rsqrt16(var + EPS)

    pltpu.sync_copy(mtmp, m_out.at[pl.ds(cid * NP + nb, NSL)])
    pltpu.sync_copy(itmp, i_out.at[pl.ds(cid * NP + nb, NSL)])
    plsc.subcore_barrier()

    g_regs = [gbuf[pl.ds(d * L, L)] for d in range(D // L)]
    b_regs = [bbuf[pl.ds(d * L, L)] for d in range(D // L)]

    def issue(c, b):
        pltpu.async_copy(m_out.at[rowv.at[c]], mbuf.at[b], semg.at[b])
        pltpu.async_copy(i_out.at[rowv.at[c]], ivbuf.at[b], semg.at[b])
        pltpu.async_copy(x_hbm.at[colv.at[c]], xbuf.at[b], semg.at[b])

    for b in range(NBF):
        issue(b, b)

    @pl.loop(0, NCH + NBF - 1, step=NBF)
    def _(g):
        for b in range(NBF):
            c = g + b

            @pl.when(c < NCH)
            def _():
                pltpu.make_async_copy(
                    m_out.at[pl.ds(0, CH)], mbuf.at[b], semg.at[b]).wait()
                pltpu.make_async_copy(
                    m_out.at[pl.ds(0, CH)], ivbuf.at[b], semg.at[b]).wait()
                pltpu.make_async_copy(
                    x_hbm.at[pl.ds(0, CH)], xbuf.at[b], semg.at[b]).wait()

                @pl.when(c >= NBF)
                def _():
                    pltpu.make_async_copy(
                        obuf.at[b], out_hbm.at[pl.ds(0, CH)], semo.at[b]).wait()

                @pl.loop(0, CH // L)
                def _(gi):
                    mvec = mbuf[b, pl.ds(gi * L, L)]
                    ivvec = ivbuf[b, pl.ds(gi * L, L)]
                    for jj in range(L):
                        j = gi * L + jj
                        m = mvec[jj]
                        iv = ivvec[jj]
                        for d in range(D // L):
                            xv = xbuf[b, j, pl.ds(d * L, L)]
                            obuf[b, j, pl.ds(d * L, L)] = (
                                (xv - m) * iv * g_regs[d] + b_regs[d])

                pltpu.async_copy(
                    obuf.at[b], out_hbm.at[pl.ds(ebase + c * CH, CH)],
                    semo.at[b])

                @pl.when(c + NBF < NCH)
                def _():
                    issue(c + NBF, b)

    for b in range(NBF):
        pltpu.make_async_copy(
            obuf.at[b], out_hbm.at[pl.ds(0, CH)], semo.at[b]).wait()


def _main_pass(x, col3, row3, parts, gamma, beta):
    mesh = plsc.VectorSubcoreMesh(core_axis_name="c", subcore_axis_name="s")
    f = pl.kernel(
        _main_body,
        out_type=[jax.ShapeDtypeStruct((E, D), jnp.float32),
                  jax.ShapeDtypeStruct((NC * NP,), jnp.float32),
                  jax.ShapeDtypeStruct((NC * NP,), jnp.float32)],
        mesh=mesh,
        scratch_types=[
            pltpu.VMEM((NCH, CH), jnp.int32),
            pltpu.VMEM((NCH, CH), jnp.int32),
            pltpu.VMEM((NBF, CH), jnp.float32),
            pltpu.VMEM((NBF, CH), jnp.float32),
            pltpu.VMEM((NBF, CH, D), jnp.float32),
            pltpu.VMEM((NBF, CH, D), jnp.float32),
            pltpu.VMEM((D,), jnp.float32),
            pltpu.VMEM((D,), jnp.float32),
            pltpu.VMEM((6, NSL), jnp.float32),
            pltpu.VMEM((NSL,), jnp.float32),
            pltpu.VMEM((NSL,), jnp.float32),
            pltpu.SemaphoreType.DMA((NBF,)),
            pltpu.SemaphoreType.DMA((NBF,)),
        ],
    )
    return f(x, col3, row3, parts, gamma, beta)[0]


@jax.jit
def kernel(x, edge_index, gamma, beta):
    ei = edge_index.astype(jnp.int32)
    row3 = ei[0].reshape(TILES, NCH, CH)
    col3 = ei[1].reshape(TILES, NCH, CH)
    bias = (jnp.arange(TILES, dtype=jnp.int32) // NS * NP)[:, None, None]
    row3b = row3 + bias
    col3b = col3 + bias

    xt = jnp.pad(x, ((0, NP - N), (0, 0))).T
    parts = _scatter_partials(xt, col3b, row3)
    return _main_pass(x, col3, row3b, parts, gamma, beta)


# speedup vs baseline: 1.0813x; 1.0002x over previous
import jax
import jax.numpy as jnp
from jax import lax
from jax.experimental import pallas as pl
from jax.experimental.pallas import tpu as pltpu
from jax.experimental.pallas import tpu_sc as plsc

EPS = 1e-05

N = 10000
D = 128
E = 320000
NP = 10240
L = 16

NC, NS = 2, 16
TILES = NC * NS
EPT = E // TILES
CH = 80
NCH = EPT // CH


def _scatter_body(xt_hbm, col_hbm, row_hbm, parts, s_out, q_out,
                  colv, rowv, sbuf, qbuf, ones, zbuf, xtt, stmp, qtmp,
                  S_sh, Q_sh, D_sh, semg, sem_sc, sem_ones):
    cid = lax.axis_index("c")
    sid = lax.axis_index("s")
    wid = cid * NS + sid

    pltpu.sync_copy(col_hbm.at[wid], colv)
    pltpu.sync_copy(row_hbm.at[wid], rowv)

    for i in range(CH // L):
        ones[pl.ds(i * L, L)] = jnp.ones((L,), jnp.float32)

    rb = sid * NSL
    rinv = jnp.float32(1.0 / D)

    for k in range(NSL // D):
        start = rb + k * D
        pltpu.sync_copy(xt_hbm.at[:, pl.ds(start, D)], xtt)

        @pl.loop(0, D // L)
        def _(gi):
            sl = pl.ds(gi * L, L)
            acc_s = jnp.zeros((L,), jnp.float32)
            acc_q = jnp.zeros((L,), jnp.float32)
            for d in range(D):
                v = xtt[d, sl]
                acc_s = acc_s + v
                acc_q = acc_q + v * v
            stmp[pl.ds(k * D + gi * L, L)] = acc_s * rinv
            qtmp[pl.ds(k * D + gi * L, L)] = acc_q * rinv

    pltpu.sync_copy(stmp, s_out.at[pl.ds(cid * NP + rb, NSL)])
    pltpu.sync_copy(qtmp, q_out.at[pl.ds(cid * NP + rb, NSL)])

    @pl.when(sid < 3)
    def _():
        @pl.loop(0, NP // L)
        def _(i):
            zbuf[pl.ds(i * L, L)] = jnp.zeros((L,), jnp.float32)

    @pl.when(sid == 0)
    def _():
        pltpu.sync_copy(zbuf, S_sh)

    @pl.when(sid == 1)
    def _():
        pltpu.sync_copy(zbuf, Q_sh)

    @pl.when(sid == 2)
    def _():
        pltpu.sync_copy(zbuf, D_sh)

    plsc.subcore_barrier()

    DEPTH = 4
    NBUF = 8

    @pl.loop(0, NCH + DEPTH)
    def _(c):
        b = lax.rem(c, NBUF)

        @pl.when(c < NCH)
        def _():
            @pl.when(c >= NBUF)
            def _():
                pltpu.make_async_copy(
                    s_out.at[pl.ds(0, CH)], sbuf.at[b], sem_sc.at[b]).wait()
                pltpu.make_async_copy(
                    s_out.at[pl.ds(0, CH)], qbuf.at[b], sem_sc.at[b]).wait()
                pltpu.make_async_copy(
                    s_out.at[pl.ds(0, CH)], ones, sem_ones).wait()

            pltpu.async_copy(s_out.at[colv.at[c]], sbuf.at[b], semg.at[b])
            pltpu.async_copy(q_out.at[colv.at[c]], qbuf.at[b], semg.at[b])

        cs = c - DEPTH

        @pl.when(cs >= 0)
        def _():
            bs = lax.rem(cs, NBUF)
            pltpu.make_async_copy(
                s_out.at[pl.ds(0, CH)], sbuf.at[bs], semg.at[bs]).wait()
            pltpu.make_async_copy(
                s_out.at[pl.ds(0, CH)], qbuf.at[bs], semg.at[bs]).wait()
            pltpu.async_copy(sbuf.at[bs], S_sh.at[rowv.at[cs]],
                             sem_sc.at[bs], add=True)
            pltpu.async_copy(qbuf.at[bs], Q_sh.at[rowv.at[cs]],
                             sem_sc.at[bs], add=True)
            pltpu.async_copy(ones, D_sh.at[rowv.at[cs]], sem_ones, add=True)

    for b in range(NBUF):
        pltpu.make_async_copy(
            s_out.at[pl.ds(0, CH)], sbuf.at[b], sem_sc.at[b]).wait()
        pltpu.make_async_copy(
            s_out.at[pl.ds(0, CH)], qbuf.at[b], sem_sc.at[b]).wait()
        pltpu.make_async_copy(
            s_out.at[pl.ds(0, CH)], ones, sem_ones).wait()

    plsc.subcore_barrier()

    @pl.when(sid == 0)
    def _():
        pltpu.sync_copy(S_sh, parts.at[pl.ds((cid * 3 + 0) * NP, NP)])
        pltpu.sync_copy(Q_sh, parts.at[pl.ds((cid * 3 + 1) * NP, NP)])
        pltpu.sync_copy(D_sh, parts.at[pl.ds((cid * 3 + 2) * NP, NP)])


def _scatter_partials(xt, col3b, row3):
    mesh = plsc.VectorSubcoreMesh(core_axis_name="c", subcore_axis_name="s")
    f = pl.kernel(
        _scatter_body,
        out_type=[jax.ShapeDtypeStruct((NC * 3 * NP,), jnp.float32),
                  jax.ShapeDtypeStruct((NC * NP,), jnp.float32),
                  jax.ShapeDtypeStruct((NC * NP,), jnp.float32)],
        mesh=mesh,
        scratch_types=[
            pltpu.VMEM((NCH, CH), jnp.int32),
            pltpu.VMEM((NCH, CH), jnp.int32),
            pltpu.VMEM((8, CH), jnp.float32),
            pltpu.VMEM((8, CH), jnp.float32),
            pltpu.VMEM((CH,), jnp.float32),
            pltpu.VMEM((NP,), jnp.float32),
            pltpu.VMEM((D, D), jnp.float32),
            pltpu.VMEM((NSL,), jnp.float32),
            pltpu.VMEM((NSL,), jnp.float32),
            pltpu.VMEM_SHARED((NP,), jnp.float32),
            pltpu.VMEM_SHARED((NP,), jnp.float32),
            pltpu.VMEM_SHARED((NP,), jnp.float32),
            pltpu.SemaphoreType.DMA((8,)),
            pltpu.SemaphoreType.DMA((8,)),
            pltpu.SemaphoreType.DMA,
        ],
    )
    return f(xt, col3b, row3)[0]


NSL = NP // NS
NBF = 3


def _rsqrt16(xx):
    i = lax.bitcast_convert_type(xx, jnp.int32)
    y = lax.bitcast_convert_type(jnp.int32(0x5F3759DF) - (i >> 1), jnp.float32)
    for _ in range(4):
        y = y * (1.5 - 0.5 * xx * y * y)
    return y


def _main_body(x_hbm, col_hbm, row_hbm, parts, g_hbm, b_hbm,
               out_hbm, m_out, i_out,
               colv, rowv, mbuf, ivbuf, xbuf, obuf, gbuf, bbuf, ptmp,
               mtmp, itmp, semg, semo):
    cid = lax.axis_index("c")
    sid = lax.axis_index("s")
    wid = cid * NS + sid
    ebase = wid * EPT

    pltpu.sync_copy(col_hbm.at[wid], colv)
    pltpu.sync_copy(row_hbm.at[wid], rowv)
    pltpu.sync_copy(g_hbm, gbuf)
    pltpu.sync_copy(b_hbm, bbuf)

    nb = sid * NSL
    for t in range(6):
        c6, t3 = divmod(t, 3)
        pltpu.sync_copy(parts.at[pl.ds((c6 * 3 + t3) * NP + nb, NSL)],
                        ptmp.at[t])

    @pl.loop(0, NSL // L)
    def _(gi):
        sl = pl.ds(gi * L, L)
        S = ptmp[0, sl] + ptmp[3, sl]
        Q = ptmp[1, sl] + ptmp[4, sl]
        dg = ptmp[2, sl] + ptmp[5, sl]
        den = jnp.maximum(dg, 1.0)
        m = S / den
        var = jnp.maximum(Q / den - m * m, 0.0)
        mtmp[sl] = m
        itmp[sl] = _rsqrt16(var + EPS)

    pltpu.sync_copy(mtmp, m_out.at[pl.ds(cid * NP + nb, NSL)])
    pltpu.sync_copy(itmp, i_out.at[pl.ds(cid * NP + nb, NSL)])
    plsc.subcore_barrier()

    g_regs = [gbuf[pl.ds(d * L, L)] for d in range(D // L)]
    b_regs = [bbuf[pl.ds(d * L, L)] for d in range(D // L)]

    def issue(c, b):
        pltpu.async_copy(m_out.at[rowv.at[c]], mbuf.at[b], semg.at[b])
        pltpu.async_copy(i_out.at[rowv.at[c]], ivbuf.at[b], semg.at[b])
        pltpu.async_copy(x_hbm.at[colv.at[c]], xbuf.at[b], semg.at[b])

    for b in range(NBF):
        issue(b, b)

    @pl.loop(0, NCH + NBF - 1, step=NBF)
    def _(g):
        for b in range(NBF):
            c = g + b

            @pl.when(c < NCH)
            def _():
                pltpu.make_async_copy(
                    m_out.at[pl.ds(0, CH)], mbuf.at[b], semg.at[b]).wait()
                pltpu.make_async_copy(
                    m_out.at[pl.ds(0, CH)], ivbuf.at[b], semg.at[b]).wait()
                pltpu.make_async_copy(
                    x_hbm.at[pl.ds(0, CH)], xbuf.at[b], semg.at[b]).wait()

                @pl.when(c >= NBF)
                def _():
                    pltpu.make_async_copy(
                        obuf.at[b], out_hbm.at[pl.ds(0, CH)], semo.at[b]).wait()

                @pl.loop(0, CH // L)
                def _(gi):
                    mvec = mbuf[b, pl.ds(gi * L, L)]
                    ivvec = ivbuf[b, pl.ds(gi * L, L)]
                    for jj in range(L):
                        j = gi * L + jj
                        m = mvec[jj]
                        iv = ivvec[jj]
                        for d in range(D // L):
                            xv = xbuf[b, j, pl.ds(d * L, L)]
                            obuf[b, j, pl.ds(d * L, L)] = (
                                (xv - m) * iv * g_regs[d] + b_regs[d])

                pltpu.async_copy(
                    obuf.at[b], out_hbm.at[pl.ds(ebase + c * CH, CH)],
                    semo.at[b])

                @pl.when(c + NBF < NCH)
                def _():
                    issue(c + NBF, b)

    for b in range(NBF):
        pltpu.make_async_copy(
            obuf.at[b], out_hbm.at[pl.ds(0, CH)], semo.at[b]).wait()


def _main_pass(x, col3, row3, parts, gamma, beta):
    mesh = plsc.VectorSubcoreMesh(core_axis_name="c", subcore_axis_name="s")
    f = pl.kernel(
        _main_body,
        out_type=[jax.ShapeDtypeStruct((E, D), jnp.float32),
                  jax.ShapeDtypeStruct((NC * NP,), jnp.float32),
                  jax.ShapeDtypeStruct((NC * NP,), jnp.float32)],
        mesh=mesh,
        scratch_types=[
            pltpu.VMEM((NCH, CH), jnp.int32),
            pltpu.VMEM((NCH, CH), jnp.int32),
            pltpu.VMEM((NBF, CH), jnp.float32),
            pltpu.VMEM((NBF, CH), jnp.float32),
            pltpu.VMEM((NBF, CH, D), jnp.float32),
            pltpu.VMEM((NBF, CH, D), jnp.float32),
            pltpu.VMEM((D,), jnp.float32),
            pltpu.VMEM((D,), jnp.float32),
            pltpu.VMEM((6, NSL), jnp.float32),
            pltpu.VMEM((NSL,), jnp.float32),
            pltpu.VMEM((NSL,), jnp.float32),
            pltpu.SemaphoreType.DMA((NBF,)),
            pltpu.SemaphoreType.DMA((NBF,)),
        ],
    )
    return f(x, col3, row3, parts, gamma, beta)[0]


@jax.jit
def kernel(x, edge_index, gamma, beta):
    ei = edge_index.astype(jnp.int32)
    row3 = ei[0].reshape(TILES, NCH, CH)
    col3 = ei[1].reshape(TILES, NCH, CH)
    bias = (jnp.arange(TILES, dtype=jnp.int32) // NS * NP)[:, None, None]
    row3b = row3 + bias
    col3b = col3 + bias

    xt = jnp.pad(x, ((0, NP - N), (0, 0))).T
    parts = _scatter_partials(xt, col3b, row3)
    return _main_pass(x, col3, row3b, parts, gamma, beta)
